# baseline (speedup 1.0000x reference)
import jax
import jax.numpy as jnp
from jax.experimental import pallas as pl
from jax.experimental.pallas import tpu as pltpu
from jax.experimental.pallas import tpu_sc as plsc

HIDDEN = 1024
FF = 4096
E = 8
TOPK = 2
BT = 256
GW = 128


def _router_body(x_ref, gw_ref, logits_ref, idx_ref, val_ref):
    x = x_ref[...].astype(jnp.bfloat16)
    gw = gw_ref[...].astype(jnp.bfloat16)
    logits = jax.lax.dot_general(
        x, gw,
        dimension_numbers=(((1,), (1,)), ((), ())),
        preferred_element_type=jnp.float32,
    )
    logits_ref[...] = logits
    rw = jax.nn.softmax(logits, axis=-1)
    idx = jax.lax.broadcasted_iota(jnp.int32, rw.shape, 1)
    v1 = jnp.max(rw, axis=1, keepdims=True)
    i1 = jnp.min(jnp.where(rw == v1, idx, E), axis=1, keepdims=True)
    masked = jnp.where(idx == i1, -jnp.inf, rw)
    v2 = jnp.max(masked, axis=1, keepdims=True)
    i2 = jnp.min(jnp.where(masked == v2, idx, E), axis=1, keepdims=True)
    denom = v1 + v2
    idx_ref[...] = jnp.concatenate([i1, i2], axis=1)
    val_ref[...] = jnp.concatenate([v1 / denom, v2 / denom], axis=1)


def _router(x, gate_w):
    t = x.shape[0]
    return pl.pallas_call(
        _router_body,
        grid=(1,),
        in_specs=[
            pl.BlockSpec((t, HIDDEN), lambda i: (0, 0)),
            pl.BlockSpec((E, HIDDEN), lambda i: (0, 0)),
        ],
        out_specs=[
            pl.BlockSpec((t, E), lambda i: (0, 0)),
            pl.BlockSpec((t, TOPK), lambda i: (0, 0)),
            pl.BlockSpec((t, TOPK), lambda i: (0, 0)),
        ],
        out_shape=[
            jax.ShapeDtypeStruct((t, E), jnp.float32),
            jax.ShapeDtypeStruct((t, TOPK), jnp.int32),
            jax.ShapeDtypeStruct((t, TOPK), jnp.float32),
        ],
    )(x, gate_w)


def _sc_gather(data, indices):
    n, w = data.shape
    n_idx = indices.shape[0]
    orig_dtype = data.dtype
    if data.dtype == jnp.bfloat16:
        d32 = jax.lax.bitcast_convert_type(
            data.reshape(n, w // 2, 2), jnp.int32)
    else:
        d32 = jax.lax.bitcast_convert_type(data, jnp.int32)
    wi = d32.shape[1]
    width = 256
    ch = wi // width
    d32 = d32.reshape(n * ch, width)
    indices = (indices[:, None] * ch
               + jnp.arange(ch, dtype=jnp.int32)[None, :]).reshape(-1)
    n_rows = indices.shape[0]
    ind = indices.reshape(1, n_rows)
    data = d32
    mesh = plsc.VectorSubcoreMesh(
        core_axis_name="core", subcore_axis_name="subcore")

    @pl.kernel(
        out_type=jax.ShapeDtypeStruct((n_rows, width), data.dtype),
        mesh=mesh)
    def k(x_hbm, i_hbm, o_hbm):
        def body(i_vmem, o_vmem):
            pltpu.sync_copy(x_hbm.at[i_vmem.at[0]], o_vmem)

        pltpu.emit_pipeline(
            body,
            grid=(n_rows // GW,),
            in_specs=[pl.BlockSpec((1, GW), lambda i: (0, i))],
            out_specs=[pl.BlockSpec((GW, width), lambda i: (i, 0))],
            core_axis_name=("core", "subcore"),
            dimension_semantics=(pltpu.PARALLEL,),
        )(i_hbm, o_hbm)

    out32 = k(data, ind).reshape(n_idx, wi)
    if orig_dtype == jnp.bfloat16:
        return jax.lax.bitcast_convert_type(
            out32, jnp.bfloat16).reshape(n_idx, w)
    return jax.lax.bitcast_convert_type(out32, orig_dtype)


def _gffn_body(be_ref, xs_ref, w1_ref, w2_ref, wt_ref, ys_ref):
    h = jax.lax.dot_general(
        xs_ref[...], w1_ref[0],
        dimension_numbers=(((1,), (0,)), ((), ())),
        preferred_element_type=jnp.float32,
    )
    h = 0.5 * h * (1.0 + jax.lax.erf(h * 0.7071067811865476))
    o = jax.lax.dot_general(
        h.astype(jnp.bfloat16), w2_ref[0],
        dimension_numbers=(((1,), (0,)), ((), ())),
        preferred_element_type=jnp.float32,
    )
    ys_ref[...] = (o * wt_ref[...]).astype(ys_ref.dtype)


def _gffn(block_e, xs, w1b, w2b, wt, p):
    nb = p // BT
    grid_spec = pltpu.PrefetchScalarGridSpec(
        num_scalar_prefetch=1,
        grid=(nb,),
        in_specs=[
            pl.BlockSpec((BT, HIDDEN), lambda i, be: (i, 0)),
            pl.BlockSpec((1, HIDDEN, FF), lambda i, be: (be[i], 0, 0)),
            pl.BlockSpec((1, FF, HIDDEN), lambda i, be: (be[i], 0, 0)),
            pl.BlockSpec((BT, 1), lambda i, be: (i, 0)),
        ],
        out_specs=pl.BlockSpec((BT, HIDDEN), lambda i, be: (i, 0)),
    )
    return pl.pallas_call(
        _gffn_body,
        grid_spec=grid_spec,
        out_shape=jax.ShapeDtypeStruct((p, HIDDEN), jnp.bfloat16),
        compiler_params=pltpu.CompilerParams(
            dimension_semantics=("arbitrary",),
        ),
    )(block_e, xs, w1b, w2b, wt)


def _combine_body(g0_ref, g1_ref, bias_ref, out_ref):
    out_ref[...] = (g0_ref[...].astype(jnp.float32)
                    + g1_ref[...].astype(jnp.float32) + bias_ref[...])


def _combine(g, bias2d, t):
    bc = 1024
    return pl.pallas_call(
        _combine_body,
        grid=(t // bc,),
        in_specs=[
            pl.BlockSpec((bc, HIDDEN), lambda i: (i, 0)),
            pl.BlockSpec((bc, HIDDEN), lambda i: (i + t // bc, 0)),
            pl.BlockSpec((1, HIDDEN), lambda i: (0, 0)),
        ],
        out_specs=pl.BlockSpec((bc, HIDDEN), lambda i: (i, 0)),
        out_shape=jax.ShapeDtypeStruct((t, HIDDEN), jnp.float32),
    )(g, g, bias2d)


def kernel(hidden_states, gate_w, w1, w2, bias):
    b, s, d = hidden_states.shape
    x = hidden_states.reshape(-1, d)
    t = x.shape[0]
    a = t * TOPK
    p = a + E * BT

    router_logits, idx2, val2 = _router(x, gate_w)

    ex = idx2.reshape(-1)
    wf = val2.reshape(-1)
    eids = jnp.arange(E, dtype=jnp.int32)
    onehot = ex[None, :] == eids[:, None]
    cum = jnp.cumsum(onehot.astype(jnp.int32), axis=1)
    rank = jnp.sum(jnp.where(onehot, cum, 0), axis=0) - 1
    counts = cum[:, -1]
    pg = ((counts + BT - 1) // BT) * BT
    pstart = jnp.concatenate(
        [jnp.zeros((1,), jnp.int32), jnp.cumsum(pg)[:-1].astype(jnp.int32)])
    p_idx = jnp.sum(jnp.where(onehot, pstart[:, None], 0), axis=0) + rank
    tok_ids = jnp.zeros((p,), jnp.int32).at[p_idx].set(
        jnp.arange(a, dtype=jnp.int32) // TOPK,
        unique_indices=True, mode="drop")
    wt = jnp.zeros((p,), jnp.float32).at[p_idx].set(
        wf, unique_indices=True, mode="drop")
    bidx = jnp.arange(p // BT, dtype=jnp.int32) * BT
    block_e = (jnp.sum((bidx[:, None] >= pstart[None, :]).astype(jnp.int32),
                       axis=1) - 1).astype(jnp.int32)
    pp = p_idx.reshape(t, TOPK)
    s_all = jnp.concatenate([pp[:, 0], pp[:, 1]])

    xb = x.astype(jnp.bfloat16)
    xs = _sc_gather(xb, tok_ids)
    ys = _gffn(block_e, xs, w1.astype(jnp.bfloat16),
               w2.astype(jnp.bfloat16), wt.reshape(p, 1), p)
    g = _sc_gather(ys, s_all)
    final = _combine(g, bias.reshape(1, HIDDEN), t)

    return (final.reshape(b, s, d), router_logits)


# speedup vs baseline: 1.3144x; 1.3144x over previous
import jax
import jax.numpy as jnp
from jax.experimental import pallas as pl
from jax.experimental.pallas import tpu as pltpu
from jax.experimental.pallas import tpu_sc as plsc

HIDDEN = 1024
FF = 4096
E = 8
TOPK = 2
BT = 256
GW = 128


def _router_body(x_ref, gw_ref, logits_ref, idx_ref, val_ref):
    x = x_ref[...].astype(jnp.bfloat16)
    gw = gw_ref[...].astype(jnp.bfloat16)
    logits = jax.lax.dot_general(
        x, gw,
        dimension_numbers=(((1,), (1,)), ((), ())),
        preferred_element_type=jnp.float32,
    )
    logits_ref[...] = logits
    rw = jax.nn.softmax(logits, axis=-1)
    idx = jax.lax.broadcasted_iota(jnp.int32, rw.shape, 1)
    v1 = jnp.max(rw, axis=1, keepdims=True)
    i1 = jnp.min(jnp.where(rw == v1, idx, E), axis=1, keepdims=True)
    masked = jnp.where(idx == i1, -jnp.inf, rw)
    v2 = jnp.max(masked, axis=1, keepdims=True)
    i2 = jnp.min(jnp.where(masked == v2, idx, E), axis=1, keepdims=True)
    denom = v1 + v2
    idx_ref[...] = jnp.concatenate([i1, i2], axis=1)
    val_ref[...] = jnp.concatenate([v1 / denom, v2 / denom], axis=1)


def _router(x, gate_w):
    t = x.shape[0]
    return pl.pallas_call(
        _router_body,
        grid=(1,),
        in_specs=[
            pl.BlockSpec((t, HIDDEN), lambda i: (0, 0)),
            pl.BlockSpec((E, HIDDEN), lambda i: (0, 0)),
        ],
        out_specs=[
            pl.BlockSpec((t, E), lambda i: (0, 0)),
            pl.BlockSpec((t, TOPK), lambda i: (0, 0)),
            pl.BlockSpec((t, TOPK), lambda i: (0, 0)),
        ],
        out_shape=[
            jax.ShapeDtypeStruct((t, E), jnp.float32),
            jax.ShapeDtypeStruct((t, TOPK), jnp.int32),
            jax.ShapeDtypeStruct((t, TOPK), jnp.float32),
        ],
    )(x, gate_w)


def _sc_gather(data, indices):
    n, w = data.shape
    n_idx = indices.shape[0]
    orig_dtype = data.dtype
    if data.dtype == jnp.bfloat16:
        d32 = jax.lax.bitcast_convert_type(
            data.reshape(n, w // 2, 2), jnp.int32)
    else:
        d32 = jax.lax.bitcast_convert_type(data, jnp.int32)
    wi = d32.shape[1]
    width = 256
    ch = wi // width
    d32 = d32.reshape(n * ch, width)
    indices = (indices[:, None] * ch
               + jnp.arange(ch, dtype=jnp.int32)[None, :]).reshape(-1)
    n_rows = indices.shape[0]
    ind = indices.reshape(1, n_rows)
    data = d32
    mesh = plsc.VectorSubcoreMesh(
        core_axis_name="core", subcore_axis_name="subcore")

    @pl.kernel(
        out_type=jax.ShapeDtypeStruct((n_rows, width), data.dtype),
        mesh=mesh)
    def k(x_hbm, i_hbm, o_hbm):
        def body(i_vmem, o_vmem):
            pltpu.sync_copy(x_hbm.at[i_vmem.at[0]], o_vmem)

        pltpu.emit_pipeline(
            body,
            grid=(n_rows // GW,),
            in_specs=[pl.BlockSpec((1, GW), lambda i: (0, i))],
            out_specs=[pl.BlockSpec((GW, width), lambda i: (i, 0))],
            core_axis_name=("core", "subcore"),
            dimension_semantics=(pltpu.PARALLEL,),
        )(i_hbm, o_hbm)

    out32 = k(data, ind).reshape(n_idx, wi)
    if orig_dtype == jnp.bfloat16:
        return jax.lax.bitcast_convert_type(
            out32, jnp.bfloat16).reshape(n_idx, w)
    return jax.lax.bitcast_convert_type(out32, orig_dtype)


def _gffn_body(s_ref, xs_ref, w1_ref, w2_ref, wt_ref, ys_ref):
    i = pl.program_id(0)
    h = jax.lax.dot_general(
        xs_ref[...], w1_ref[0],
        dimension_numbers=(((1,), (0,)), ((), ())),
        preferred_element_type=jnp.float32,
    )
    h = 0.5 * h * (1.0 + jax.lax.erf(h * 0.7071067811865476))
    o = jax.lax.dot_general(
        h.astype(jnp.bfloat16), w2_ref[0],
        dimension_numbers=(((1,), (0,)), ((), ())),
        preferred_element_type=jnp.float32,
    )
    r = (jax.lax.broadcasted_iota(jnp.int32, (BT, 1), 0)
         + s_ref[0, i] * BT)
    mask = ((r >= s_ref[2, i]) & (r < s_ref[3, i])).astype(jnp.float32)
    contrib = (o * (wt_ref[...] * mask)).astype(ys_ref.dtype)

    @pl.when(s_ref[4, i] == 1)
    def _():
        ys_ref[...] = contrib

    @pl.when(s_ref[4, i] == 0)
    def _():
        ys_ref[...] += contrib


def _gffn(sinfo, xs, w1b, w2b, wt, nw):
    grid_spec = pltpu.PrefetchScalarGridSpec(
        num_scalar_prefetch=1,
        grid=(nw,),
        in_specs=[
            pl.BlockSpec((BT, HIDDEN), lambda i, s: (s[0, i], 0)),
            pl.BlockSpec((1, HIDDEN, FF), lambda i, s: (s[1, i], 0, 0)),
            pl.BlockSpec((1, FF, HIDDEN), lambda i, s: (s[1, i], 0, 0)),
            pl.BlockSpec((BT, 1), lambda i, s: (s[0, i], 0)),
        ],
        out_specs=pl.BlockSpec((BT, HIDDEN), lambda i, s: (s[0, i], 0)),
    )
    return pl.pallas_call(
        _gffn_body,
        grid_spec=grid_spec,
        out_shape=jax.ShapeDtypeStruct((xs.shape[0], HIDDEN), jnp.bfloat16),
        compiler_params=pltpu.CompilerParams(
            dimension_semantics=("arbitrary",),
        ),
    )(sinfo, xs, w1b, w2b, wt)


def _combine_body(g0_ref, g1_ref, bias_ref, out_ref):
    out_ref[...] = (g0_ref[...].astype(jnp.float32)
                    + g1_ref[...].astype(jnp.float32) + bias_ref[...])


def _combine(g, bias2d, t):
    bc = 1024
    return pl.pallas_call(
        _combine_body,
        grid=(t // bc,),
        in_specs=[
            pl.BlockSpec((bc, HIDDEN), lambda i: (i, 0)),
            pl.BlockSpec((bc, HIDDEN), lambda i: (i + t // bc, 0)),
            pl.BlockSpec((1, HIDDEN), lambda i: (0, 0)),
        ],
        out_specs=pl.BlockSpec((bc, HIDDEN), lambda i: (i, 0)),
        out_shape=jax.ShapeDtypeStruct((t, HIDDEN), jnp.float32),
    )(g, g, bias2d)


def kernel(hidden_states, gate_w, w1, w2, bias):
    b, s, d = hidden_states.shape
    x = hidden_states.reshape(-1, d)
    t = x.shape[0]
    a = t * TOPK
    nbc = a // BT
    nw = nbc + E

    router_logits, idx2, val2 = _router(x, gate_w)

    ex = idx2.reshape(-1)
    wf = val2.reshape(-1)
    aid = jnp.arange(a, dtype=jnp.int32)
    _, a_s, wf_s = jax.lax.sort((ex, aid, wf), num_keys=1)
    tok = a_s // TOPK
    _, inv_j = jax.lax.sort((a_s, aid), num_keys=1)
    pp = inv_j.reshape(t, TOPK)
    s_all = jnp.concatenate([pp[:, 0], pp[:, 1]])

    eids = jnp.arange(E, dtype=jnp.int32)
    counts = jnp.sum((ex[None, :] == eids[:, None]).astype(jnp.int32), axis=1)
    cend = jnp.cumsum(counts)
    cstart = cend - counts
    bpos = jnp.arange(nbc, dtype=jnp.int32) * BT
    elo = jnp.sum((cend[None, :] <= bpos[:, None]).astype(jnp.int32), axis=1)
    ehi = jnp.sum((cend[None, :] <= (bpos + BT - 1)[:, None]).astype(jnp.int32),
                  axis=1)
    nspan = ehi - elo + 1
    start = jnp.cumsum(nspan) - nspan
    total = start[-1] + nspan[-1]
    jidx = jnp.arange(nw, dtype=jnp.int32)
    blk = jnp.sum((start[None, :] <= jidx[:, None]).astype(jnp.int32),
                  axis=1) - 1
    ohb = (blk[:, None] == jnp.arange(nbc, dtype=jnp.int32)[None, :])
    blk_start = jnp.sum(jnp.where(ohb, start[None, :], 0), axis=1)
    blk_elo = jnp.sum(jnp.where(ohb, elo[None, :], 0), axis=1)
    blk_ehi = jnp.sum(jnp.where(ohb, ehi[None, :], 0), axis=1)
    eix = jnp.minimum(blk_elo + (jidx - blk_start), blk_ehi)
    valid = jidx < total
    ohe = (eix[:, None] == eids[None, :])
    lo = jnp.where(valid,
                   jnp.sum(jnp.where(ohe, cstart[None, :], 0), axis=1), 0)
    hi = jnp.where(valid,
                   jnp.sum(jnp.where(ohe, cend[None, :], 0), axis=1), 0)
    first = (jidx == blk_start).astype(jnp.int32)
    sinfo = jnp.stack([blk, eix, lo, hi, first]).astype(jnp.int32)

    xb = x.astype(jnp.bfloat16)
    xs = _sc_gather(xb, tok)
    ys = _gffn(sinfo, xs, w1.astype(jnp.bfloat16),
               w2.astype(jnp.bfloat16), wf_s.reshape(a, 1), nw)
    g = _sc_gather(ys, s_all)
    final = _combine(g, bias.reshape(1, HIDDEN), t)

    return (final.reshape(b, s, d), router_logits)


# speedup vs baseline: 1.3186x; 1.0033x over previous
import jax
import jax.numpy as jnp
from jax.experimental import pallas as pl
from jax.experimental.pallas import tpu as pltpu
from jax.experimental.pallas import tpu_sc as plsc

HIDDEN = 1024
FF = 4096
E = 8
TOPK = 2
BT = 256
GW = 128


def _router_body(x_ref, gw_ref, logits_ref, idx_ref, val_ref):
    x = x_ref[...].astype(jnp.bfloat16)
    gw = gw_ref[...].astype(jnp.bfloat16)
    logits = jax.lax.dot_general(
        x, gw,
        dimension_numbers=(((1,), (1,)), ((), ())),
        preferred_element_type=jnp.float32,
    )
    logits_ref[...] = logits
    rw = jax.nn.softmax(logits, axis=-1)
    idx = jax.lax.broadcasted_iota(jnp.int32, rw.shape, 1)
    v1 = jnp.max(rw, axis=1, keepdims=True)
    i1 = jnp.min(jnp.where(rw == v1, idx, E), axis=1, keepdims=True)
    masked = jnp.where(idx == i1, -jnp.inf, rw)
    v2 = jnp.max(masked, axis=1, keepdims=True)
    i2 = jnp.min(jnp.where(masked == v2, idx, E), axis=1, keepdims=True)
    denom = v1 + v2
    idx_ref[...] = jnp.concatenate([i1, i2], axis=1)
    val_ref[...] = jnp.concatenate([v1 / denom, v2 / denom], axis=1)


def _router(x, gate_w):
    t = x.shape[0]
    return pl.pallas_call(
        _router_body,
        grid=(1,),
        in_specs=[
            pl.BlockSpec((t, HIDDEN), lambda i: (0, 0)),
            pl.BlockSpec((E, HIDDEN), lambda i: (0, 0)),
        ],
        out_specs=[
            pl.BlockSpec((t, E), lambda i: (0, 0)),
            pl.BlockSpec((t, TOPK), lambda i: (0, 0)),
            pl.BlockSpec((t, TOPK), lambda i: (0, 0)),
        ],
        out_shape=[
            jax.ShapeDtypeStruct((t, E), jnp.float32),
            jax.ShapeDtypeStruct((t, TOPK), jnp.int32),
            jax.ShapeDtypeStruct((t, TOPK), jnp.float32),
        ],
    )(x, gate_w)


def _sc_gather(data, indices):
    n, w = data.shape
    n_idx = indices.shape[0]
    orig_dtype = data.dtype
    if data.dtype == jnp.bfloat16:
        d32 = jax.lax.bitcast_convert_type(
            data.reshape(n, w // 2, 2), jnp.int32)
    else:
        d32 = jax.lax.bitcast_convert_type(data, jnp.int32)
    wi = d32.shape[1]
    width = 256
    ch = wi // width
    d32 = d32.reshape(n * ch, width)
    indices = (indices[:, None] * ch
               + jnp.arange(ch, dtype=jnp.int32)[None, :]).reshape(-1)
    n_rows = indices.shape[0]
    ind = indices.reshape(1, n_rows)
    data = d32
    mesh = plsc.VectorSubcoreMesh(
        core_axis_name="core", subcore_axis_name="subcore")

    @pl.kernel(
        out_type=jax.ShapeDtypeStruct((n_rows, width), data.dtype),
        mesh=mesh)
    def k(x_hbm, i_hbm, o_hbm):
        def body(i_vmem, o_vmem):
            pltpu.sync_copy(x_hbm.at[i_vmem.at[0]], o_vmem)

        pltpu.emit_pipeline(
            body,
            grid=(n_rows // GW,),
            in_specs=[pl.BlockSpec((1, GW), lambda i: (0, i))],
            out_specs=[pl.BlockSpec((GW, width), lambda i: (i, 0))],
            core_axis_name=("core", "subcore"),
            dimension_semantics=(pltpu.PARALLEL,),
        )(i_hbm, o_hbm)

    out32 = k(data, ind).reshape(n_idx, wi)
    if orig_dtype == jnp.bfloat16:
        return jax.lax.bitcast_convert_type(
            out32, jnp.bfloat16).reshape(n_idx, w)
    return jax.lax.bitcast_convert_type(out32, orig_dtype)


def _gffn_body(s_ref, xs_ref, w1_ref, w2_ref, wt_ref, ys_ref):
    i = pl.program_id(0)
    h = jax.lax.dot_general(
        xs_ref[...], w1_ref[0],
        dimension_numbers=(((1,), (0,)), ((), ())),
        preferred_element_type=jnp.float32,
    )
    h = 0.5 * h * (1.0 + jax.lax.erf(h * 0.7071067811865476))
    o = jax.lax.dot_general(
        h.astype(jnp.bfloat16), w2_ref[0],
        dimension_numbers=(((1,), (0,)), ((), ())),
        preferred_element_type=jnp.float32,
    )
    r = (jax.lax.broadcasted_iota(jnp.int32, (BT, 1), 0)
         + s_ref[0, i] * BT)
    mask = ((r >= s_ref[2, i]) & (r < s_ref[3, i])).astype(jnp.float32)
    contrib = (o * (wt_ref[...] * mask)).astype(ys_ref.dtype)

    @pl.when(s_ref[4, i] == 1)
    def _():
        ys_ref[...] = contrib

    @pl.when(s_ref[4, i] == 0)
    def _():
        ys_ref[...] += contrib


def _gffn(sinfo, xs, w1b, w2b, wt, nw):
    grid_spec = pltpu.PrefetchScalarGridSpec(
        num_scalar_prefetch=1,
        grid=(nw,),
        in_specs=[
            pl.BlockSpec((BT, HIDDEN), lambda i, s: (s[0, i], 0)),
            pl.BlockSpec((1, HIDDEN, FF), lambda i, s: (s[1, i], 0, 0)),
            pl.BlockSpec((1, FF, HIDDEN), lambda i, s: (s[1, i], 0, 0)),
            pl.BlockSpec((BT, 1), lambda i, s: (s[0, i], 0)),
        ],
        out_specs=pl.BlockSpec((BT, HIDDEN), lambda i, s: (s[0, i], 0)),
    )
    return pl.pallas_call(
        _gffn_body,
        grid_spec=grid_spec,
        out_shape=jax.ShapeDtypeStruct((xs.shape[0], HIDDEN), jnp.bfloat16),
        compiler_params=pltpu.CompilerParams(
            dimension_semantics=("arbitrary",),
        ),
    )(sinfo, xs, w1b, w2b, wt)


def _cast_body(w1_ref, w2_ref, o1_ref, o2_ref):
    o1_ref[...] = w1_ref[...].astype(jnp.bfloat16)
    o2_ref[...] = w2_ref[...].astype(jnp.bfloat16)


def _cast_weights(w1, w2):
    return pl.pallas_call(
        _cast_body,
        grid=(2 * E,),
        in_specs=[
            pl.BlockSpec((1, HIDDEN // 2, FF), lambda i: (i // 2, i % 2, 0)),
            pl.BlockSpec((1, FF // 2, HIDDEN), lambda i: (i // 2, i % 2, 0)),
        ],
        out_specs=[
            pl.BlockSpec((1, HIDDEN // 2, FF), lambda i: (i // 2, i % 2, 0)),
            pl.BlockSpec((1, FF // 2, HIDDEN), lambda i: (i // 2, i % 2, 0)),
        ],
        out_shape=[
            jax.ShapeDtypeStruct(w1.shape, jnp.bfloat16),
            jax.ShapeDtypeStruct(w2.shape, jnp.bfloat16),
        ],
        compiler_params=pltpu.CompilerParams(
            dimension_semantics=("arbitrary",),
        ),
    )(w1, w2)


def _combine_body(g0_ref, g1_ref, bias_ref, out_ref):
    out_ref[...] = (g0_ref[...].astype(jnp.float32)
                    + g1_ref[...].astype(jnp.float32) + bias_ref[...])


def _combine(g, bias2d, t):
    bc = 1024
    return pl.pallas_call(
        _combine_body,
        grid=(t // bc,),
        in_specs=[
            pl.BlockSpec((bc, HIDDEN), lambda i: (i, 0)),
            pl.BlockSpec((bc, HIDDEN), lambda i: (i + t // bc, 0)),
            pl.BlockSpec((1, HIDDEN), lambda i: (0, 0)),
        ],
        out_specs=pl.BlockSpec((bc, HIDDEN), lambda i: (i, 0)),
        out_shape=jax.ShapeDtypeStruct((t, HIDDEN), jnp.float32),
    )(g, g, bias2d)


def kernel(hidden_states, gate_w, w1, w2, bias):
    b, s, d = hidden_states.shape
    x = hidden_states.reshape(-1, d)
    t = x.shape[0]
    a = t * TOPK
    nbc = a // BT
    nw = nbc + E

    router_logits, idx2, val2 = _router(x, gate_w)

    ex = idx2.reshape(-1)
    wf = val2.reshape(-1)
    aid = jnp.arange(a, dtype=jnp.int32)
    _, a_s, wf_s = jax.lax.sort((ex, aid, wf), num_keys=1)
    tok = a_s // TOPK
    _, inv_j = jax.lax.sort((a_s, aid), num_keys=1)
    pp = inv_j.reshape(t, TOPK)
    s_all = jnp.concatenate([pp[:, 0], pp[:, 1]])

    eids = jnp.arange(E, dtype=jnp.int32)
    counts = jnp.sum((ex[None, :] == eids[:, None]).astype(jnp.int32), axis=1)
    cend = jnp.cumsum(counts)
    cstart = cend - counts
    bpos = jnp.arange(nbc, dtype=jnp.int32) * BT
    elo = jnp.sum((cend[None, :] <= bpos[:, None]).astype(jnp.int32), axis=1)
    ehi = jnp.sum((cend[None, :] <= (bpos + BT - 1)[:, None]).astype(jnp.int32),
                  axis=1)
    nspan = ehi - elo + 1
    start = jnp.cumsum(nspan) - nspan
    total = start[-1] + nspan[-1]
    jidx = jnp.arange(nw, dtype=jnp.int32)
    blk = jnp.sum((start[None, :] <= jidx[:, None]).astype(jnp.int32),
                  axis=1) - 1
    ohb = (blk[:, None] == jnp.arange(nbc, dtype=jnp.int32)[None, :])
    blk_start = jnp.sum(jnp.where(ohb, start[None, :], 0), axis=1)
    blk_elo = jnp.sum(jnp.where(ohb, elo[None, :], 0), axis=1)
    blk_ehi = jnp.sum(jnp.where(ohb, ehi[None, :], 0), axis=1)
    eix = jnp.minimum(blk_elo + (jidx - blk_start), blk_ehi)
    valid = jidx < total
    ohe = (eix[:, None] == eids[None, :])
    lo = jnp.where(valid,
                   jnp.sum(jnp.where(ohe, cstart[None, :], 0), axis=1), 0)
    hi = jnp.where(valid,
                   jnp.sum(jnp.where(ohe, cend[None, :], 0), axis=1), 0)
    first = (jidx == blk_start).astype(jnp.int32)
    sinfo = jnp.stack([blk, eix, lo, hi, first]).astype(jnp.int32)

    xb = x.astype(jnp.bfloat16)
    w1b, w2b = _cast_weights(w1, w2)
    xs = _sc_gather(xb, tok)
    ys = _gffn(sinfo, xs, w1b, w2b, wf_s.reshape(a, 1), nw)
    g = _sc_gather(ys, s_all)
    final = _combine(g, bias.reshape(1, HIDDEN), t)

    return (final.reshape(b, s, d), router_logits)


# speedup vs baseline: 1.7638x; 1.3376x over previous
import jax
import jax.numpy as jnp
from jax.experimental import pallas as pl
from jax.experimental.pallas import tpu as pltpu

HIDDEN = 1024
FF = 4096
E = 8
TOPK = 2
BT = 512


def _router_body(x_ref, gw_ref, logits_ref, w_ref, xb_ref):
    x = x_ref[...].astype(jnp.bfloat16)
    xb_ref[...] = x
    gw = gw_ref[...].astype(jnp.bfloat16)
    logits = jax.lax.dot_general(
        x, gw,
        dimension_numbers=(((1,), (1,)), ((), ())),
        preferred_element_type=jnp.float32,
    )
    logits_ref[...] = logits
    rw = jax.nn.softmax(logits, axis=-1)
    idx = jax.lax.broadcasted_iota(jnp.int32, rw.shape, 1)
    v1 = jnp.max(rw, axis=1, keepdims=True)
    i1 = jnp.min(jnp.where(rw == v1, idx, E), axis=1, keepdims=True)
    masked = jnp.where(idx == i1, -jnp.inf, rw)
    v2 = jnp.max(masked, axis=1, keepdims=True)
    i2 = jnp.min(jnp.where(masked == v2, idx, E), axis=1, keepdims=True)
    denom = v1 + v2
    w = jnp.where(idx == i1, v1, 0.0) + jnp.where(idx == i2, v2, 0.0)
    w_ref[...] = w / denom


def _router(x, gate_w):
    t = x.shape[0]
    return pl.pallas_call(
        _router_body,
        grid=(1,),
        in_specs=[
            pl.BlockSpec((t, HIDDEN), lambda i: (0, 0)),
            pl.BlockSpec((E, HIDDEN), lambda i: (0, 0)),
        ],
        out_specs=[
            pl.BlockSpec((t, E), lambda i: (0, 0)),
            pl.BlockSpec((t, E), lambda i: (0, 0)),
            pl.BlockSpec((t, HIDDEN), lambda i: (0, 0)),
        ],
        out_shape=[
            jax.ShapeDtypeStruct((t, E), jnp.float32),
            jax.ShapeDtypeStruct((t, E), jnp.float32),
            jax.ShapeDtypeStruct((t, HIDDEN), jnp.bfloat16),
        ],
    )(x, gate_w)


def _cast_body(w1_ref, w2_ref, o1_ref, o2_ref):
    o1_ref[...] = w1_ref[...].astype(jnp.bfloat16)
    o2_ref[...] = w2_ref[...].astype(jnp.bfloat16)


def _cast_weights(w1, w2):
    return pl.pallas_call(
        _cast_body,
        grid=(2 * E,),
        in_specs=[
            pl.BlockSpec((1, HIDDEN // 2, FF), lambda i: (i // 2, i % 2, 0)),
            pl.BlockSpec((1, FF // 2, HIDDEN), lambda i: (i // 2, i % 2, 0)),
        ],
        out_specs=[
            pl.BlockSpec((1, HIDDEN // 2, FF), lambda i: (i // 2, i % 2, 0)),
            pl.BlockSpec((1, FF // 2, HIDDEN), lambda i: (i // 2, i % 2, 0)),
        ],
        out_shape=[
            jax.ShapeDtypeStruct(w1.shape, jnp.bfloat16),
            jax.ShapeDtypeStruct(w2.shape, jnp.bfloat16),
        ],
        compiler_params=pltpu.CompilerParams(
            dimension_semantics=("arbitrary",),
        ),
    )(w1, w2)


def _ffn_body(x_ref, w1_ref, w2_ref, wts_ref, bias_ref, out_ref):
    e = pl.program_id(1)

    @pl.when(e == 0)
    def _():
        out_ref[...] = jnp.broadcast_to(bias_ref[...], out_ref.shape)

    h = jax.lax.dot_general(
        x_ref[...], w1_ref[0],
        dimension_numbers=(((1,), (0,)), ((), ())),
        preferred_element_type=jnp.float32,
    )
    h = 0.5 * h * (1.0 + jax.lax.erf(h * 0.7071067811865476))
    o = jax.lax.dot_general(
        h.astype(jnp.bfloat16), w2_ref[0],
        dimension_numbers=(((1,), (0,)), ((), ())),
        preferred_element_type=jnp.float32,
    )
    out_ref[...] += o * wts_ref[0]


def _ffn(xb, w1b, w2b, wts, bias2d):
    t = xb.shape[0]
    grid = (t // BT, E)
    return pl.pallas_call(
        _ffn_body,
        grid=grid,
        in_specs=[
            pl.BlockSpec((BT, HIDDEN), lambda i, e: (i, 0)),
            pl.BlockSpec((1, HIDDEN, FF), lambda i, e: (e, 0, 0)),
            pl.BlockSpec((1, FF, HIDDEN), lambda i, e: (e, 0, 0)),
            pl.BlockSpec((1, BT, 1), lambda i, e: (e, i, 0)),
            pl.BlockSpec((1, HIDDEN), lambda i, e: (0, 0)),
        ],
        out_specs=pl.BlockSpec((BT, HIDDEN), lambda i, e: (i, 0)),
        out_shape=jax.ShapeDtypeStruct((t, HIDDEN), jnp.float32),
        compiler_params=pltpu.CompilerParams(
            dimension_semantics=("parallel", "arbitrary"),
        ),
    )(xb, w1b, w2b, wts, bias2d)


def kernel(hidden_states, gate_w, w1, w2, bias):
    b, s, d = hidden_states.shape
    x = hidden_states.reshape(-1, d)
    t = x.shape[0]

    router_logits, wmat, xb = _router(x, gate_w)
    w1b, w2b = _cast_weights(w1, w2)

    wts = wmat.T.reshape(E, t, 1)
    bias2d = bias.reshape(1, HIDDEN)

    final = _ffn(xb, w1b, w2b, wts, bias2d)
    return (final.reshape(b, s, d), router_logits)


# speedup vs baseline: 1.7710x; 1.0041x over previous
import jax
import jax.numpy as jnp
from jax.experimental import pallas as pl
from jax.experimental.pallas import tpu as pltpu

HIDDEN = 1024
FF = 4096
E = 8
TOPK = 2
BT = 512


def _router_body(x_ref, gw_ref, logits_ref, w_ref, xb_ref):
    x = x_ref[...].astype(jnp.bfloat16)
    xb_ref[...] = x
    gw = gw_ref[...].astype(jnp.bfloat16)
    logits = jax.lax.dot_general(
        x, gw,
        dimension_numbers=(((1,), (1,)), ((), ())),
        preferred_element_type=jnp.float32,
    )
    logits_ref[...] = logits
    rw = jax.nn.softmax(logits, axis=-1)
    idx = jax.lax.broadcasted_iota(jnp.int32, rw.shape, 1)
    v1 = jnp.max(rw, axis=1, keepdims=True)
    i1 = jnp.min(jnp.where(rw == v1, idx, E), axis=1, keepdims=True)
    masked = jnp.where(idx == i1, -jnp.inf, rw)
    v2 = jnp.max(masked, axis=1, keepdims=True)
    i2 = jnp.min(jnp.where(masked == v2, idx, E), axis=1, keepdims=True)
    denom = v1 + v2
    w = jnp.where(idx == i1, v1, 0.0) + jnp.where(idx == i2, v2, 0.0)
    w_ref[...] = w / denom


def _router(x, gate_w):
    t = x.shape[0]
    return pl.pallas_call(
        _router_body,
        grid=(1,),
        in_specs=[
            pl.BlockSpec((t, HIDDEN), lambda i: (0, 0)),
            pl.BlockSpec((E, HIDDEN), lambda i: (0, 0)),
        ],
        out_specs=[
            pl.BlockSpec((t, E), lambda i: (0, 0)),
            pl.BlockSpec((t, E), lambda i: (0, 0)),
            pl.BlockSpec((t, HIDDEN), lambda i: (0, 0)),
        ],
        out_shape=[
            jax.ShapeDtypeStruct((t, E), jnp.float32),
            jax.ShapeDtypeStruct((t, E), jnp.float32),
            jax.ShapeDtypeStruct((t, HIDDEN), jnp.bfloat16),
        ],
    )(x, gate_w)


def _cast_body(w1_ref, w2_ref, o1_ref, o2_ref):
    o1_ref[...] = w1_ref[...].astype(jnp.bfloat16)
    o2_ref[...] = w2_ref[...].astype(jnp.bfloat16)


def _cast_weights(w1, w2):
    return pl.pallas_call(
        _cast_body,
        grid=(2 * E,),
        in_specs=[
            pl.BlockSpec((1, HIDDEN // 2, FF), lambda i: (i // 2, i % 2, 0)),
            pl.BlockSpec((1, FF // 2, HIDDEN), lambda i: (i // 2, i % 2, 0)),
        ],
        out_specs=[
            pl.BlockSpec((1, HIDDEN // 2, FF), lambda i: (i // 2, i % 2, 0)),
            pl.BlockSpec((1, FF // 2, HIDDEN), lambda i: (i // 2, i % 2, 0)),
        ],
        out_shape=[
            jax.ShapeDtypeStruct(w1.shape, jnp.bfloat16),
            jax.ShapeDtypeStruct(w2.shape, jnp.bfloat16),
        ],
        compiler_params=pltpu.CompilerParams(
            dimension_semantics=("arbitrary",),
        ),
    )(w1, w2)


def _ffn_body(x_ref, w1_ref, w2_ref, wts_ref, bias_ref, out_ref):
    e = pl.program_id(1)

    @pl.when(e == 0)
    def _():
        out_ref[...] = jnp.broadcast_to(bias_ref[...], out_ref.shape)

    h = jax.lax.dot_general(
        x_ref[...], w1_ref[0],
        dimension_numbers=(((1,), (0,)), ((), ())),
        preferred_element_type=jnp.float32,
    )
    hb = h.astype(jnp.bfloat16)
    gb = hb * (0.5 + 0.5 * jax.lax.erf(hb * jnp.bfloat16(0.70710678)))
    o = jax.lax.dot_general(
        gb, w2_ref[0],
        dimension_numbers=(((1,), (0,)), ((), ())),
        preferred_element_type=jnp.float32,
    )
    out_ref[...] += o * wts_ref[0]


def _ffn(xb, w1b, w2b, wts, bias2d):
    t = xb.shape[0]
    grid = (t // BT, E)
    return pl.pallas_call(
        _ffn_body,
        grid=grid,
        in_specs=[
            pl.BlockSpec((BT, HIDDEN), lambda i, e: (i, 0)),
            pl.BlockSpec((1, HIDDEN, FF), lambda i, e: (e, 0, 0)),
            pl.BlockSpec((1, FF, HIDDEN), lambda i, e: (e, 0, 0)),
            pl.BlockSpec((1, BT, 1), lambda i, e: (e, i, 0)),
            pl.BlockSpec((1, HIDDEN), lambda i, e: (0, 0)),
        ],
        out_specs=pl.BlockSpec((BT, HIDDEN), lambda i, e: (i, 0)),
        out_shape=jax.ShapeDtypeStruct((t, HIDDEN), jnp.float32),
        compiler_params=pltpu.CompilerParams(
            dimension_semantics=("parallel", "arbitrary"),
        ),
    )(xb, w1b, w2b, wts, bias2d)


def kernel(hidden_states, gate_w, w1, w2, bias):
    b, s, d = hidden_states.shape
    x = hidden_states.reshape(-1, d)
    t = x.shape[0]

    router_logits, wmat, xb = _router(x, gate_w)
    w1b, w2b = _cast_weights(w1, w2)

    wts = wmat.T.reshape(E, t, 1)
    bias2d = bias.reshape(1, HIDDEN)

    final = _ffn(xb, w1b, w2b, wts, bias2d)
    return (final.reshape(b, s, d), router_logits)
